# Initial kernel scaffold; baseline (speedup 1.0000x reference)
import jax
import jax.numpy as jnp
from jax.experimental import pallas as pl


def kernel(x, edge_index, edge_type, bases1, wcomp1, loop1, bias1, bases2, wcomp2, loop2, bias2):
    raise NotImplementedError("write your pallas kernel here")


# speedup vs baseline: 3.2221x; 3.2221x over previous
import functools

import jax
import jax.numpy as jnp
from jax import lax
from jax.experimental import pallas as pl
from jax.experimental.pallas import tpu as pltpu
from jax.experimental.pallas import tpu_sc as plsc

N = 10000
E = 160000
D = 256
H = 256
R = 8
NB_BASES = 4
NR = N * R

HALF = 128
NTILES = 16
EPT = E // NTILES
PBLK = 2000
KB = 80
BN = 1000


def _sc_prep(src, dst, typ, z128):
    mesh = plsc.VectorSubcoreMesh(core_axis_name="c", subcore_axis_name="s")
    DROWS = 640

    @functools.partial(
        pl.kernel,
        out_type=(jax.ShapeDtypeStruct((E,), jnp.float32),
                  jax.ShapeDtypeStruct((E,), jnp.int32),
                  jax.ShapeDtypeStruct((DROWS, 128), jnp.float32)),
        mesh=mesh,
        compiler_params=pltpu.CompilerParams(needs_layout_passes=False,
                                             use_tc_tiling_on_sc=True),
        scratch_types=[
            pltpu.VMEM((PBLK,), jnp.int32),
            pltpu.VMEM((PBLK,), jnp.int32),
            pltpu.VMEM((PBLK,), jnp.int32),
            pltpu.VMEM((PBLK,), jnp.float32),
            pltpu.VMEM((PBLK,), jnp.int32),
            pltpu.VMEM((1, KB), jnp.int32),
            pltpu.VMEM((KB, 128), jnp.float32),
            pltpu.VMEM((KB * 128,), jnp.float32),
            pltpu.VMEM_SHARED((DROWS, 128), jnp.float32),
            pltpu.SemaphoreType.DMA,
        ],
    )
    def prep(src_h, dst_h, typ_h, z128_h, w_h, gi_h, degh_h,
             dst_st, typ_st, src_st, w_st, gi_st, seg_b, dv, dflat,
             deg_sh, sem):
        cid = lax.axis_index("c")
        sid = lax.axis_index("s")
        tbase = sid * EPT
        iota16 = lax.iota(jnp.int32, 16)

        @pl.when(sid < 10)
        def _():
            pltpu.sync_copy(z128_h.at[pl.ds(0, DROWS // 10)],
                            deg_sh.at[pl.ds(sid * (DROWS // 10), DROWS // 10)])
        plsc.subcore_barrier()

        for blk in range(EPT // PBLK):
            b0 = tbase + blk * PBLK
            pltpu.sync_copy(dst_h.at[pl.ds(b0, PBLK)], dst_st)
            pltpu.sync_copy(typ_h.at[pl.ds(b0, PBLK)], typ_st)

            def suba(k2, carry):
                e0 = k2 * KB
                for c in range(KB // 16):
                    sl = pl.ds(e0 + c * 16, 16)
                    seg = dst_st[sl] * R + typ_st[sl]
                    seg_b[0, pl.ds(c * 16, 16)] = lax.shift_right_logical(
                        seg, 7)
                    lanes = jnp.bitwise_and(seg, 127)
                    for j in range(16):
                        lane_b = lanes.at[jnp.full((16,), j, jnp.int32)].get(
                            mode="promise_in_bounds")
                        for cc in range(8):
                            dv[c * 16 + j, pl.ds(cc * 16, 16)] = jnp.where(
                                iota16 + cc * 16 == lane_b, 1.0, 0.0)
                pltpu.sync_copy(dv, deg_sh.at[seg_b.at[0]], add=True)
                return carry
            lax.fori_loop(0, PBLK // KB, suba, None)
        plsc.subcore_barrier()

        @pl.when(sid < 10)
        def _():
            pltpu.sync_copy(deg_sh.at[pl.ds(sid * (DROWS // 10), DROWS // 10)],
                            degh_h.at[pl.ds(sid * (DROWS // 10), DROWS // 10)])
        plsc.subcore_barrier()

        for blk in range(EPT // PBLK):
            b0 = tbase + blk * PBLK
            pltpu.sync_copy(dst_h.at[pl.ds(b0, PBLK)], dst_st)
            pltpu.sync_copy(typ_h.at[pl.ds(b0, PBLK)], typ_st)
            pltpu.sync_copy(src_h.at[pl.ds(b0, PBLK)], src_st)

            def subb(k2, carry):
                e0 = k2 * KB
                for c in range(KB // 16):
                    sl = pl.ds(e0 + c * 16, 16)
                    seg = dst_st[sl] * R + typ_st[sl]
                    seg_b[0, pl.ds(c * 16, 16)] = lax.shift_right_logical(
                        seg, 7)
                pltpu.async_copy(degh_h.at[seg_b.at[0]], dv, sem).wait()
                for rr in range(KB):
                    for cc in range(8):
                        dflat[pl.ds(rr * 128 + cc * 16, 16)] = (
                            dv[rr, pl.ds(cc * 16, 16)])
                for c in range(KB // 16):
                    sl = pl.ds(e0 + c * 16, 16)
                    seg = dst_st[sl] * R + typ_st[sl]
                    pos = ((iota16 + c * 16) * 128
                           + jnp.bitwise_and(seg, 127))
                    dvv = plsc.load_gather(dflat, [pos])
                    w_st[sl] = 1.0 / jnp.maximum(dvv, 1.0)
                    gi_st[sl] = typ_st[sl] * N + src_st[sl]
                return carry
            lax.fori_loop(0, PBLK // KB, subb, None)

            @pl.when(cid == 0)
            def _():
                pltpu.sync_copy(w_st, w_h.at[pl.ds(b0, PBLK)])
                pltpu.sync_copy(gi_st, gi_h.at[pl.ds(b0, PBLK)])

    return prep(src, dst, typ, z128)


def _sc_aggregate(gidx, dst, w, y, z128):
    mesh = plsc.VectorSubcoreMesh(core_axis_name="c", subcore_axis_name="s")
    slab = 624
    tail = N - NTILES * slab

    @functools.partial(
        pl.kernel,
        out_type=jax.ShapeDtypeStruct((2, N, HALF), jnp.float32),
        mesh=mesh,
        compiler_params=pltpu.CompilerParams(needs_layout_passes=False,
                                             use_tc_tiling_on_sc=True),
        scratch_types=[
            pltpu.VMEM((PBLK,), jnp.int32),
            pltpu.VMEM((PBLK,), jnp.int32),
            pltpu.VMEM((PBLK,), jnp.float32),
            pltpu.VMEM((1, KB), jnp.int32),
            pltpu.VMEM((1, KB), jnp.int32),
            pltpu.VMEM((KB, HALF), jnp.float32),
            pltpu.VMEM_SHARED((N, HALF), jnp.float32),
            pltpu.SemaphoreType.DMA,
        ],
    )
    def agg(gi_h, dst_h, w_h, y_h, z128_h, out_h,
            gi_st, dst_st, w_st, gi_b, di_b, msg, acc_sh, sem):
        cid = lax.axis_index("c")
        sid = lax.axis_index("s")
        tbase = sid * EPT

        r0 = sid * slab
        pltpu.sync_copy(z128_h.at[pl.ds(0, slab)], acc_sh.at[pl.ds(r0, slab)])

        @pl.when(sid == 0)
        def _():
            pltpu.sync_copy(z128_h.at[pl.ds(0, tail)],
                            acc_sh.at[pl.ds(NTILES * slab, tail)])
        plsc.subcore_barrier()

        coff = cid * (R * N)

        for mb in range(EPT // PBLK):
            b0 = tbase + mb * PBLK
            pltpu.sync_copy(gi_h.at[pl.ds(b0, PBLK)], gi_st)
            pltpu.sync_copy(dst_h.at[pl.ds(b0, PBLK)], dst_st)
            pltpu.sync_copy(w_h.at[pl.ds(b0, PBLK)], w_st)

            def blk(i, carry):
                e0 = i * KB
                for c in range(KB // 16):
                    sl = pl.ds(e0 + c * 16, 16)
                    sb = pl.ds(c * 16, 16)
                    gi_b[0, sb] = gi_st[sl] + coff
                    di_b[0, sb] = dst_st[sl]
                pltpu.async_copy(y_h.at[gi_b.at[0]], msg, sem).wait()
                for c in range(KB // 16):
                    wv16 = w_st[pl.ds(e0 + c * 16, 16)]
                    for j in range(16):
                        wbj = wv16.at[jnp.full((16,), j, jnp.int32)].get(
                            mode="promise_in_bounds")
                        row = c * 16 + j
                        for cc in range(HALF // 16):
                            s2 = pl.ds(cc * 16, 16)
                            msg[row, s2] = msg[row, s2] * wbj
                pltpu.sync_copy(msg, acc_sh.at[di_b.at[0]], add=True)
                return carry
            lax.fori_loop(0, PBLK // KB, blk, None)

        plsc.subcore_barrier()
        pltpu.sync_copy(acc_sh.at[pl.ds(r0, slab)],
                        out_h.at[cid, pl.ds(r0, slab)])

        @pl.when(sid == 0)
        def _():
            pltpu.sync_copy(acc_sh.at[pl.ds(NTILES * slab, tail)],
                            out_h.at[cid, pl.ds(NTILES * slab, tail)])

    return agg(gidx, dst, w, y, z128)


def _tc_transform(h, wcomp, bases):
    def body(x_ref, wc_ref, b_ref, y_ref, wscr):
        i = pl.program_id(0)

        @pl.when(i == 0)
        def _():
            wc = wc_ref[...]
            ba = b_ref[...]
            for r in range(R):
                acc = wc[r, 0] * ba[0]
                for b in range(1, NB_BASES):
                    acc = acc + wc[r, b] * ba[b]
                wscr[r] = acc

        xb = x_ref[...]
        for r in range(R):
            y = jnp.dot(xb, wscr[r], preferred_element_type=jnp.float32)
            y_ref[0, r] = y[:, :HALF]
            y_ref[1, r] = y[:, HALF:]

    return pl.pallas_call(
        body,
        grid=(N // BN,),
        in_specs=[
            pl.BlockSpec((BN, D), lambda i: (i, 0)),
            pl.BlockSpec((R, NB_BASES), lambda i: (0, 0)),
            pl.BlockSpec((NB_BASES, D, H), lambda i: (0, 0, 0)),
        ],
        out_specs=pl.BlockSpec((2, R, BN, HALF), lambda i: (0, 0, i, 0)),
        out_shape=jax.ShapeDtypeStruct((2, R, N, HALF), jnp.float32),
        scratch_shapes=[pltpu.VMEM((R, D, H), jnp.float32)],
    )(h, wcomp, bases)


def _tc_finish(acc2, h, loop_w, bias2d):
    def body(a_ref, x_ref, l_ref, bias_ref, o_ref):
        a = jnp.concatenate([a_ref[0], a_ref[1]], axis=-1)
        out = a + jnp.dot(x_ref[...], l_ref[...],
                          preferred_element_type=jnp.float32) + bias_ref[...]
        o_ref[...] = jnp.maximum(out, 0.0)

    return pl.pallas_call(
        body,
        grid=(N // BN,),
        in_specs=[
            pl.BlockSpec((2, BN, HALF), lambda i: (0, i, 0)),
            pl.BlockSpec((BN, D), lambda i: (i, 0)),
            pl.BlockSpec((D, H), lambda i: (0, 0)),
            pl.BlockSpec((1, H), lambda i: (0, 0)),
        ],
        out_specs=pl.BlockSpec((BN, H), lambda i: (i, 0)),
        out_shape=jax.ShapeDtypeStruct((N, H), jnp.float32),
    )(acc2, h, loop_w, bias2d)


def kernel(x, edge_index, edge_type, bases1, wcomp1, loop1, bias1,
           bases2, wcomp2, loop2, bias2):
    src = edge_index[0]
    dst = edge_index[1]
    z128 = jnp.zeros((640, HALF), jnp.float32)
    w, gidx, _ = _sc_prep(src, dst, edge_type, z128)

    h = x
    for bases, wcomp, loop_w, bias in ((bases1, wcomp1, loop1, bias1),
                                       (bases2, wcomp2, loop2, bias2)):
        y = _tc_transform(h, wcomp, bases).reshape(2 * R * N, HALF)
        acc2 = _sc_aggregate(gidx, dst, w, y, z128)
        h = _tc_finish(acc2, h, loop_w, bias.reshape(1, H))
    return h


# speedup vs baseline: 4.3186x; 1.3403x over previous
import functools

import jax
import jax.numpy as jnp
from jax import lax
from jax.experimental import pallas as pl
from jax.experimental.pallas import tpu as pltpu
from jax.experimental.pallas import tpu_sc as plsc

N = 10000
E = 160000
D = 256
H = 256
R = 8
NB_BASES = 4
NR = N * R

HALF = 128
NTILES = 16
EPT = E // NTILES
PBLK = 2000
KB = 80
BN = 1000


def _sc_prep(src, dst, typ, z128):
    mesh = plsc.VectorSubcoreMesh(core_axis_name="c", subcore_axis_name="s")
    DROWS = 640

    @functools.partial(
        pl.kernel,
        out_type=(jax.ShapeDtypeStruct((E,), jnp.float32),
                  jax.ShapeDtypeStruct((E,), jnp.int32),
                  jax.ShapeDtypeStruct((DROWS, 128), jnp.float32)),
        mesh=mesh,
        compiler_params=pltpu.CompilerParams(needs_layout_passes=False,
                                             use_tc_tiling_on_sc=True),
        scratch_types=[
            pltpu.VMEM((PBLK,), jnp.int32),
            pltpu.VMEM((PBLK,), jnp.int32),
            pltpu.VMEM((PBLK,), jnp.int32),
            pltpu.VMEM((PBLK,), jnp.float32),
            pltpu.VMEM((PBLK,), jnp.int32),
            pltpu.VMEM((1, KB), jnp.int32),
            pltpu.VMEM((KB, 128), jnp.float32),
            pltpu.VMEM_SHARED((640, 128), jnp.float32),
            pltpu.SemaphoreType.DMA,
        ],
    )
    def prep(src_h, dst_h, typ_h, z128_h, w_h, gi_h, degh_h,
             dst_st, typ_st, src_st, w_st, gi_st, seg_b, dv, deg_sh, sem):
        cid = lax.axis_index("c")
        sid = lax.axis_index("s")
        tbase = sid * EPT
        iota16 = lax.iota(jnp.int32, 16)
        ones16 = jnp.ones((16,), jnp.float32)
        zeros16 = jnp.zeros((16,), jnp.float32)

        pltpu.sync_copy(z128_h.at[pl.ds(0, KB)], dv)

        @pl.when(sid < 10)
        def _():
            pltpu.sync_copy(z128_h.at[pl.ds(0, DROWS // 10)],
                            deg_sh.at[pl.ds(sid * (DROWS // 10),
                                            DROWS // 10)])
        plsc.subcore_barrier()

        for blk in range(EPT // PBLK):
            b0 = tbase + blk * PBLK
            pltpu.sync_copy(dst_h.at[pl.ds(b0, PBLK)], dst_st)
            pltpu.sync_copy(typ_h.at[pl.ds(b0, PBLK)], typ_st)

            def suba(k2, carry):
                e0 = k2 * KB
                lanes_l = []
                for c in range(KB // 16):
                    sl = pl.ds(e0 + c * 16, 16)
                    seg = dst_st[sl] * R + typ_st[sl]
                    seg_b[0, pl.ds(c * 16, 16)] = lax.shift_right_logical(
                        seg, 7)
                    lanes = jnp.bitwise_and(seg, 127)
                    lanes_l.append(lanes)
                    plsc.store_scatter(dv, [iota16 + c * 16, lanes], ones16)
                pltpu.sync_copy(dv, deg_sh.at[seg_b.at[0]], add=True)
                for c in range(KB // 16):
                    plsc.store_scatter(dv, [iota16 + c * 16, lanes_l[c]],
                                       zeros16)
                return carry
            lax.fori_loop(0, PBLK // KB, suba, None)
        plsc.subcore_barrier()

        @pl.when(sid < 10)
        def _():
            r0 = sid * (DROWS // 10)
            pltpu.sync_copy(deg_sh.at[pl.ds(r0, DROWS // 10)],
                            dv.at[pl.ds(0, DROWS // 10)])
            for j in range(DROWS // 10):
                for cc in range(8):
                    sl = pl.ds(cc * 16, 16)
                    dv[j, sl] = 1.0 / jnp.maximum(dv[j, sl], 1.0)
            pltpu.sync_copy(dv.at[pl.ds(0, DROWS // 10)],
                            degh_h.at[pl.ds(r0, DROWS // 10)])
        plsc.subcore_barrier()

        for blk in range(EPT // PBLK):
            b0 = tbase + blk * PBLK
            pltpu.sync_copy(dst_h.at[pl.ds(b0, PBLK)], dst_st)
            pltpu.sync_copy(typ_h.at[pl.ds(b0, PBLK)], typ_st)
            pltpu.sync_copy(src_h.at[pl.ds(b0, PBLK)], src_st)

            def subb(k2, carry):
                e0 = k2 * KB
                for c in range(KB // 16):
                    sl = pl.ds(e0 + c * 16, 16)
                    seg = dst_st[sl] * R + typ_st[sl]
                    seg_b[0, pl.ds(c * 16, 16)] = lax.shift_right_logical(
                        seg, 7)
                pltpu.async_copy(degh_h.at[seg_b.at[0]], dv, sem).wait()
                for c in range(KB // 16):
                    sl = pl.ds(e0 + c * 16, 16)
                    seg = dst_st[sl] * R + typ_st[sl]
                    w_st[sl] = plsc.load_gather(
                        dv, [iota16 + c * 16, jnp.bitwise_and(seg, 127)])
                    gi_st[sl] = typ_st[sl] * N + src_st[sl]
                return carry
            lax.fori_loop(0, PBLK // KB, subb, None)

            @pl.when(cid == 0)
            def _():
                pltpu.sync_copy(w_st, w_h.at[pl.ds(b0, PBLK)])
                pltpu.sync_copy(gi_st, gi_h.at[pl.ds(b0, PBLK)])

    return prep(src, dst, typ, z128)


def _sc_aggregate(gidx, dst, w, y, z128):
    mesh = plsc.VectorSubcoreMesh(core_axis_name="c", subcore_axis_name="s")
    slab = 624
    tail = N - NTILES * slab
    NSUB = PBLK // KB

    @functools.partial(
        pl.kernel,
        out_type=jax.ShapeDtypeStruct((2, N, HALF), jnp.float32),
        mesh=mesh,
        compiler_params=pltpu.CompilerParams(needs_layout_passes=False,
                                             use_tc_tiling_on_sc=True),
        scratch_types=[
            pltpu.VMEM((PBLK,), jnp.int32),
            pltpu.VMEM((PBLK,), jnp.int32),
            pltpu.VMEM((PBLK,), jnp.float32),
            pltpu.VMEM((2, KB), jnp.int32),
            pltpu.VMEM((2, KB), jnp.int32),
            pltpu.VMEM((2, KB, HALF), jnp.float32),
            pltpu.VMEM_SHARED((N, HALF), jnp.float32),
            pltpu.SemaphoreType.DMA,
            pltpu.SemaphoreType.DMA,
            pltpu.SemaphoreType.DMA,
            pltpu.SemaphoreType.DMA,
        ],
    )
    def agg(gi_h, dst_h, w_h, y_h, z128_h, out_h,
            gi_st, dst_st, w_st, gi_b, di_b, msg, acc_sh,
            sg0, sg1, ss0, ss1):
        cid = lax.axis_index("c")
        sid = lax.axis_index("s")
        tbase = sid * EPT
        sg = (sg0, sg1)
        ss = (ss0, ss1)

        r0 = sid * slab
        pltpu.sync_copy(z128_h.at[pl.ds(0, slab)], acc_sh.at[pl.ds(r0, slab)])

        @pl.when(sid == 0)
        def _():
            pltpu.sync_copy(z128_h.at[pl.ds(0, tail)],
                            acc_sh.at[pl.ds(NTILES * slab, tail)])
        plsc.subcore_barrier()

        coff = cid * (R * N)

        def build(i, b):
            e0 = i * KB
            for c in range(KB // 16):
                sl = pl.ds(e0 + c * 16, 16)
                sb = pl.ds(c * 16, 16)
                gi_b[b, sb] = gi_st[sl] + coff
                di_b[b, sb] = dst_st[sl]

        def start_gather(b):
            pltpu.make_async_copy(y_h.at[gi_b.at[b]], msg.at[b],
                                  sg[b]).start()

        def wait_gather(b):
            pltpu.make_async_copy(y_h.at[gi_b.at[b]], msg.at[b],
                                  sg[b]).wait()

        def start_scatter(b):
            pltpu.async_copy(msg.at[b], acc_sh.at[di_b.at[b]], ss[b],
                             add=True)

        def wait_scatter(b):
            pltpu.make_async_copy(msg.at[b], acc_sh.at[di_b.at[b]],
                                  ss[b]).wait()

        def scale(i, b):
            def srow(j, carry):
                idxv = lax.broadcast_in_dim(i * KB + j, (16,), ())
                wbj = plsc.load_gather(w_st, [idxv])
                for cc in range(HALF // 16):
                    s2 = pl.ds(cc * 16, 16)
                    msg[b, j, s2] = msg[b, j, s2] * wbj
                return carry
            lax.fori_loop(0, KB, srow, None)

        for mb in range(EPT // PBLK):
            b0 = tbase + mb * PBLK
            pltpu.sync_copy(gi_h.at[pl.ds(b0, PBLK)], gi_st)
            pltpu.sync_copy(dst_h.at[pl.ds(b0, PBLK)], dst_st)
            pltpu.sync_copy(w_h.at[pl.ds(b0, PBLK)], w_st)

            build(0, 0)
            start_gather(0)

            def pair(i2, carry):
                for b2 in (0, 1):
                    i = i2 * 2 + b2
                    nxt = i + 1

                    @pl.when(i >= 1)
                    def _():
                        wait_scatter(1 - b2)

                    @pl.when(nxt <= NSUB - 1)
                    def _():
                        build(nxt, 1 - b2)
                        start_gather(1 - b2)
                    wait_gather(b2)
                    scale(i, b2)
                    start_scatter(b2)
                return carry
            lax.fori_loop(0, (NSUB - 1) // 2, pair, None)

            wait_scatter(1)
            wait_gather(0)
            scale(NSUB - 1, 0)
            start_scatter(0)
            wait_scatter(0)

        plsc.subcore_barrier()
        pltpu.sync_copy(acc_sh.at[pl.ds(r0, slab)],
                        out_h.at[cid, pl.ds(r0, slab)])

        @pl.when(sid == 0)
        def _():
            pltpu.sync_copy(acc_sh.at[pl.ds(NTILES * slab, tail)],
                            out_h.at[cid, pl.ds(NTILES * slab, tail)])

    return agg(gidx, dst, w, y, z128)


def _tc_transform(h, wcomp, bases):
    def body(x_ref, wc_ref, b_ref, y_ref, wscr):
        i = pl.program_id(0)

        @pl.when(i == 0)
        def _():
            wc = wc_ref[...]
            ba = b_ref[...]
            for r in range(R):
                acc = wc[r, 0] * ba[0]
                for b in range(1, NB_BASES):
                    acc = acc + wc[r, b] * ba[b]
                wscr[r] = acc

        xb = x_ref[...]
        for r in range(R):
            y = jnp.dot(xb, wscr[r], preferred_element_type=jnp.float32)
            y_ref[0, r] = y[:, :HALF]
            y_ref[1, r] = y[:, HALF:]

    return pl.pallas_call(
        body,
        grid=(N // BN,),
        in_specs=[
            pl.BlockSpec((BN, D), lambda i: (i, 0)),
            pl.BlockSpec((R, NB_BASES), lambda i: (0, 0)),
            pl.BlockSpec((NB_BASES, D, H), lambda i: (0, 0, 0)),
        ],
        out_specs=pl.BlockSpec((2, R, BN, HALF), lambda i: (0, 0, i, 0)),
        out_shape=jax.ShapeDtypeStruct((2, R, N, HALF), jnp.float32),
        scratch_shapes=[pltpu.VMEM((R, D, H), jnp.float32)],
    )(h, wcomp, bases)


def _tc_finish(acc2, h, loop_w, bias2d):
    def body(a_ref, x_ref, l_ref, bias_ref, o_ref):
        a = jnp.concatenate([a_ref[0], a_ref[1]], axis=-1)
        out = a + jnp.dot(x_ref[...], l_ref[...],
                          preferred_element_type=jnp.float32) + bias_ref[...]
        o_ref[...] = jnp.maximum(out, 0.0)

    return pl.pallas_call(
        body,
        grid=(N // BN,),
        in_specs=[
            pl.BlockSpec((2, BN, HALF), lambda i: (0, i, 0)),
            pl.BlockSpec((BN, D), lambda i: (i, 0)),
            pl.BlockSpec((D, H), lambda i: (0, 0)),
            pl.BlockSpec((1, H), lambda i: (0, 0)),
        ],
        out_specs=pl.BlockSpec((BN, H), lambda i: (i, 0)),
        out_shape=jax.ShapeDtypeStruct((N, H), jnp.float32),
    )(acc2, h, loop_w, bias2d)


def kernel(x, edge_index, edge_type, bases1, wcomp1, loop1, bias1,
           bases2, wcomp2, loop2, bias2):
    src = edge_index[0]
    dst = edge_index[1]
    z128 = jnp.zeros((640, HALF), jnp.float32)
    w, gidx, _ = _sc_prep(src, dst, edge_type, z128)

    h = x
    for bases, wcomp, loop_w, bias in ((bases1, wcomp1, loop1, bias1),
                                       (bases2, wcomp2, loop2, bias2)):
        y = _tc_transform(h, wcomp, bases).reshape(2 * R * N, HALF)
        acc2 = _sc_aggregate(gidx, dst, w, y, z128)
        h = _tc_finish(acc2, h, loop_w, bias.reshape(1, H))
    return h


# speedup vs baseline: 4.4933x; 1.0404x over previous
import functools

import jax
import jax.numpy as jnp
from jax import lax
from jax.experimental import pallas as pl
from jax.experimental.pallas import tpu as pltpu
from jax.experimental.pallas import tpu_sc as plsc

N = 10000
E = 160000
D = 256
H = 256
R = 8
NB_BASES = 4
NR = N * R

HALF = 128
NTILES = 16
EPT = E // NTILES
PBLK = 2000
KB = 80
BN = 1000


def _sc_prep(src, dst, typ, z128):
    mesh = plsc.VectorSubcoreMesh(core_axis_name="c", subcore_axis_name="s")
    DROWS = 640

    @functools.partial(
        pl.kernel,
        out_type=(jax.ShapeDtypeStruct((E,), jnp.float32),
                  jax.ShapeDtypeStruct((E,), jnp.int32),
                  jax.ShapeDtypeStruct((DROWS, 128), jnp.float32)),
        mesh=mesh,
        compiler_params=pltpu.CompilerParams(needs_layout_passes=False,
                                             use_tc_tiling_on_sc=True),
        scratch_types=[
            pltpu.VMEM((PBLK,), jnp.int32),
            pltpu.VMEM((PBLK,), jnp.int32),
            pltpu.VMEM((PBLK,), jnp.int32),
            pltpu.VMEM((PBLK,), jnp.float32),
            pltpu.VMEM((PBLK,), jnp.int32),
            pltpu.VMEM((1, KB), jnp.int32),
            pltpu.VMEM((KB, 128), jnp.float32),
            pltpu.VMEM((640, 128), jnp.float32),
            pltpu.VMEM((5, 128), jnp.int32),
            pltpu.VMEM_SHARED((640, 128), jnp.float32),
            pltpu.SemaphoreType.DMA,
        ],
    )
    def prep(src_h, dst_h, typ_h, z128_h, w_h, gi_h, degh_h,
             dst_st, typ_st, src_st, w_st, gi_st, seg_b, dv, hist, rowi,
             deg_sh, sem):
        cid = lax.axis_index("c")
        sid = lax.axis_index("s")
        tbase = sid * EPT
        iota16 = lax.iota(jnp.int32, 16)
        ones16 = jnp.ones((16,), jnp.float32)

        pltpu.sync_copy(z128_h, hist)

        @pl.when(sid < 10)
        def _():
            pltpu.sync_copy(z128_h.at[pl.ds(0, DROWS // 10)],
                            deg_sh.at[pl.ds(sid * (DROWS // 10),
                                            DROWS // 10)])

        for blk in range(EPT // PBLK):
            b0 = tbase + blk * PBLK
            pltpu.sync_copy(dst_h.at[pl.ds(b0, PBLK)], dst_st)
            pltpu.sync_copy(typ_h.at[pl.ds(b0, PBLK)], typ_st)

            def suba(k2, carry):
                sl = pl.ds(k2 * 16, 16)
                seg = dst_st[sl] * R + typ_st[sl]
                plsc.addupdate_scatter(
                    hist,
                    [lax.shift_right_logical(seg, 7),
                     jnp.bitwise_and(seg, 127)],
                    ones16)
                return carry
            lax.fori_loop(0, PBLK // 16, suba, None)
        for p in range(5):
            for cc in range(8):
                rowi[p, pl.ds(cc * 16, 16)] = iota16 + (p * 128 + cc * 16)
        plsc.subcore_barrier()
        for p in range(5):
            pltpu.sync_copy(hist.at[pl.ds(p * 128, 128)],
                            deg_sh.at[rowi.at[p]], add=True)
        plsc.subcore_barrier()

        @pl.when(sid < 10)
        def _():
            r0 = sid * (DROWS // 10)
            pltpu.sync_copy(deg_sh.at[pl.ds(r0, DROWS // 10)],
                            dv.at[pl.ds(0, DROWS // 10)])
            for j in range(DROWS // 10):
                for cc in range(8):
                    sl = pl.ds(cc * 16, 16)
                    dv[j, sl] = 1.0 / jnp.maximum(dv[j, sl], 1.0)
            pltpu.sync_copy(dv.at[pl.ds(0, DROWS // 10)],
                            degh_h.at[pl.ds(r0, DROWS // 10)])
        plsc.subcore_barrier()

        for blk in range(EPT // PBLK):
            b0 = tbase + blk * PBLK
            pltpu.sync_copy(dst_h.at[pl.ds(b0, PBLK)], dst_st)
            pltpu.sync_copy(typ_h.at[pl.ds(b0, PBLK)], typ_st)
            pltpu.sync_copy(src_h.at[pl.ds(b0, PBLK)], src_st)

            def subb(k2, carry):
                e0 = k2 * KB
                for c in range(KB // 16):
                    sl = pl.ds(e0 + c * 16, 16)
                    seg = dst_st[sl] * R + typ_st[sl]
                    seg_b[0, pl.ds(c * 16, 16)] = lax.shift_right_logical(
                        seg, 7)
                pltpu.async_copy(degh_h.at[seg_b.at[0]], dv, sem).wait()
                for c in range(KB // 16):
                    sl = pl.ds(e0 + c * 16, 16)
                    seg = dst_st[sl] * R + typ_st[sl]
                    w_st[sl] = plsc.load_gather(
                        dv, [iota16 + c * 16, jnp.bitwise_and(seg, 127)])
                    gi_st[sl] = typ_st[sl] * N + src_st[sl]
                return carry
            lax.fori_loop(0, PBLK // KB, subb, None)

            @pl.when(cid == 0)
            def _():
                pltpu.sync_copy(w_st, w_h.at[pl.ds(b0, PBLK)])
                pltpu.sync_copy(gi_st, gi_h.at[pl.ds(b0, PBLK)])

    return prep(src, dst, typ, z128)


def _sc_aggregate(gidx, dst, w, y, z128):
    mesh = plsc.VectorSubcoreMesh(core_axis_name="c", subcore_axis_name="s")
    slab = 624
    tail = N - NTILES * slab
    NSUB = PBLK // KB

    @functools.partial(
        pl.kernel,
        out_type=jax.ShapeDtypeStruct((2, N, HALF), jnp.float32),
        mesh=mesh,
        compiler_params=pltpu.CompilerParams(needs_layout_passes=False,
                                             use_tc_tiling_on_sc=True),
        scratch_types=[
            pltpu.VMEM((PBLK,), jnp.int32),
            pltpu.VMEM((PBLK,), jnp.int32),
            pltpu.VMEM((PBLK,), jnp.float32),
            pltpu.VMEM((2, KB), jnp.int32),
            pltpu.VMEM((2, KB), jnp.int32),
            pltpu.VMEM((2, KB, HALF), jnp.float32),
            pltpu.VMEM_SHARED((N, HALF), jnp.float32),
            pltpu.SemaphoreType.DMA,
            pltpu.SemaphoreType.DMA,
            pltpu.SemaphoreType.DMA,
            pltpu.SemaphoreType.DMA,
        ],
    )
    def agg(gi_h, dst_h, w_h, y_h, z128_h, out_h,
            gi_st, dst_st, w_st, gi_b, di_b, msg, acc_sh,
            sg0, sg1, ss0, ss1):
        cid = lax.axis_index("c")
        sid = lax.axis_index("s")
        tbase = sid * EPT
        sg = (sg0, sg1)
        ss = (ss0, ss1)

        r0 = sid * slab
        pltpu.sync_copy(z128_h.at[pl.ds(0, slab)], acc_sh.at[pl.ds(r0, slab)])

        @pl.when(sid == 0)
        def _():
            pltpu.sync_copy(z128_h.at[pl.ds(0, tail)],
                            acc_sh.at[pl.ds(NTILES * slab, tail)])
        plsc.subcore_barrier()

        coff = cid * (R * N)

        def build(i, b):
            e0 = i * KB
            for c in range(KB // 16):
                sl = pl.ds(e0 + c * 16, 16)
                sb = pl.ds(c * 16, 16)
                gi_b[b, sb] = gi_st[sl] + coff
                di_b[b, sb] = dst_st[sl]

        def start_gather(b):
            pltpu.make_async_copy(y_h.at[gi_b.at[b]], msg.at[b],
                                  sg[b]).start()

        def wait_gather(b):
            pltpu.make_async_copy(y_h.at[gi_b.at[b]], msg.at[b],
                                  sg[b]).wait()

        def start_scatter(b):
            pltpu.async_copy(msg.at[b], acc_sh.at[di_b.at[b]], ss[b],
                             add=True)

        def wait_scatter(b):
            pltpu.make_async_copy(msg.at[b], acc_sh.at[di_b.at[b]],
                                  ss[b]).wait()

        def scale(i, b):
            def srow(j, carry):
                idxv = lax.broadcast_in_dim(i * KB + j, (16,), ())
                wbj = plsc.load_gather(w_st, [idxv])
                for cc in range(HALF // 16):
                    s2 = pl.ds(cc * 16, 16)
                    msg[b, j, s2] = msg[b, j, s2] * wbj
                return carry
            lax.fori_loop(0, KB, srow, None)

        for mb in range(EPT // PBLK):
            b0 = tbase + mb * PBLK
            pltpu.sync_copy(gi_h.at[pl.ds(b0, PBLK)], gi_st)
            pltpu.sync_copy(dst_h.at[pl.ds(b0, PBLK)], dst_st)
            pltpu.sync_copy(w_h.at[pl.ds(b0, PBLK)], w_st)

            build(0, 0)
            start_gather(0)

            def pair(i2, carry):
                for b2 in (0, 1):
                    i = i2 * 2 + b2
                    nxt = i + 1

                    @pl.when(i >= 1)
                    def _():
                        wait_scatter(1 - b2)

                    @pl.when(nxt <= NSUB - 1)
                    def _():
                        build(nxt, 1 - b2)
                        start_gather(1 - b2)
                    wait_gather(b2)
                    scale(i, b2)
                    start_scatter(b2)
                return carry
            lax.fori_loop(0, (NSUB - 1) // 2, pair, None)

            wait_scatter(1)
            wait_gather(0)
            scale(NSUB - 1, 0)
            start_scatter(0)
            wait_scatter(0)

        plsc.subcore_barrier()
        pltpu.sync_copy(acc_sh.at[pl.ds(r0, slab)],
                        out_h.at[cid, pl.ds(r0, slab)])

        @pl.when(sid == 0)
        def _():
            pltpu.sync_copy(acc_sh.at[pl.ds(NTILES * slab, tail)],
                            out_h.at[cid, pl.ds(NTILES * slab, tail)])

    return agg(gidx, dst, w, y, z128)


def _tc_transform(h, wcomp, bases):
    def body(x_ref, wc_ref, b_ref, y_ref, wscr):
        i = pl.program_id(0)

        @pl.when(i == 0)
        def _():
            wc = wc_ref[...]
            ba = b_ref[...]
            for r in range(R):
                acc = wc[r, 0] * ba[0]
                for b in range(1, NB_BASES):
                    acc = acc + wc[r, b] * ba[b]
                wscr[r] = acc

        xb = x_ref[...]
        for r in range(R):
            y = jnp.dot(xb, wscr[r], preferred_element_type=jnp.float32)
            y_ref[0, r] = y[:, :HALF]
            y_ref[1, r] = y[:, HALF:]

    return pl.pallas_call(
        body,
        grid=(N // BN,),
        in_specs=[
            pl.BlockSpec((BN, D), lambda i: (i, 0)),
            pl.BlockSpec((R, NB_BASES), lambda i: (0, 0)),
            pl.BlockSpec((NB_BASES, D, H), lambda i: (0, 0, 0)),
        ],
        out_specs=pl.BlockSpec((2, R, BN, HALF), lambda i: (0, 0, i, 0)),
        out_shape=jax.ShapeDtypeStruct((2, R, N, HALF), jnp.float32),
        scratch_shapes=[pltpu.VMEM((R, D, H), jnp.float32)],
    )(h, wcomp, bases)


def _tc_finish(acc2, h, loop_w, bias2d):
    def body(a_ref, x_ref, l_ref, bias_ref, o_ref):
        a = jnp.concatenate([a_ref[0], a_ref[1]], axis=-1)
        out = a + jnp.dot(x_ref[...], l_ref[...],
                          preferred_element_type=jnp.float32) + bias_ref[...]
        o_ref[...] = jnp.maximum(out, 0.0)

    return pl.pallas_call(
        body,
        grid=(N // BN,),
        in_specs=[
            pl.BlockSpec((2, BN, HALF), lambda i: (0, i, 0)),
            pl.BlockSpec((BN, D), lambda i: (i, 0)),
            pl.BlockSpec((D, H), lambda i: (0, 0)),
            pl.BlockSpec((1, H), lambda i: (0, 0)),
        ],
        out_specs=pl.BlockSpec((BN, H), lambda i: (i, 0)),
        out_shape=jax.ShapeDtypeStruct((N, H), jnp.float32),
    )(acc2, h, loop_w, bias2d)


def kernel(x, edge_index, edge_type, bases1, wcomp1, loop1, bias1,
           bases2, wcomp2, loop2, bias2):
    src = edge_index[0]
    dst = edge_index[1]
    z128 = jnp.zeros((640, HALF), jnp.float32)
    w, gidx, _ = _sc_prep(src, dst, edge_type, z128)

    h = x
    for bases, wcomp, loop_w, bias in ((bases1, wcomp1, loop1, bias1),
                                       (bases2, wcomp2, loop2, bias2)):
        y = _tc_transform(h, wcomp, bases).reshape(2 * R * N, HALF)
        acc2 = _sc_aggregate(gidx, dst, w, y, z128)
        h = _tc_finish(acc2, h, loop_w, bias.reshape(1, H))
    return h


# speedup vs baseline: 4.6546x; 1.0359x over previous
import functools

import jax
import jax.numpy as jnp
from jax import lax
from jax.experimental import pallas as pl
from jax.experimental.pallas import tpu as pltpu
from jax.experimental.pallas import tpu_sc as plsc

N = 10000
E = 160000
D = 256
H = 256
R = 8
NB_BASES = 4
NR = N * R

HALF = 128
NTILES = 16
EPT = E // NTILES
PBLK = 2000
KB = 80
BN = 1000


def _sc_prep(src, dst, typ, z128):
    mesh = plsc.VectorSubcoreMesh(core_axis_name="c", subcore_axis_name="s")
    DROWS = 640

    @functools.partial(
        pl.kernel,
        out_type=(jax.ShapeDtypeStruct((E,), jnp.float32),
                  jax.ShapeDtypeStruct((E,), jnp.int32),
                  jax.ShapeDtypeStruct((DROWS, 128), jnp.float32)),
        mesh=mesh,
        compiler_params=pltpu.CompilerParams(needs_layout_passes=False,
                                             use_tc_tiling_on_sc=True),
        scratch_types=[
            pltpu.VMEM((PBLK,), jnp.int32),
            pltpu.VMEM((PBLK,), jnp.int32),
            pltpu.VMEM((PBLK,), jnp.int32),
            pltpu.VMEM((PBLK,), jnp.float32),
            pltpu.VMEM((PBLK,), jnp.int32),
            pltpu.VMEM((1, KB), jnp.int32),
            pltpu.VMEM((1, KB), jnp.int32),
            pltpu.VMEM((KB, 128), jnp.float32),
            pltpu.VMEM((KB, 128), jnp.float32),
            pltpu.VMEM((640, 128), jnp.float32),
            pltpu.VMEM((5, 128), jnp.int32),
            pltpu.VMEM_SHARED((640, 128), jnp.float32),
            pltpu.SemaphoreType.DMA,
            pltpu.SemaphoreType.DMA,
        ],
    )
    def prep(src_h, dst_h, typ_h, z128_h, w_h, gi_h, degh_h,
             dst_st, typ_st, src_st, w_st, gi_st, seg_a, seg_c, dva, dvb,
             hist, rowi, deg_sh, sd0, sd1):
        cid = lax.axis_index("c")
        sid = lax.axis_index("s")
        tbase = sid * EPT
        iota16 = lax.iota(jnp.int32, 16)
        ones16 = jnp.ones((16,), jnp.float32)

        pltpu.sync_copy(z128_h, hist)

        @pl.when(sid < 10)
        def _():
            pltpu.sync_copy(z128_h.at[pl.ds(0, DROWS // 10)],
                            deg_sh.at[pl.ds(sid * (DROWS // 10),
                                            DROWS // 10)])

        for blk in range(EPT // PBLK):
            b0 = tbase + blk * PBLK
            pltpu.sync_copy(dst_h.at[pl.ds(b0, PBLK)], dst_st)
            pltpu.sync_copy(typ_h.at[pl.ds(b0, PBLK)], typ_st)

            def suba(k2, carry):
                sl = pl.ds(k2 * 16, 16)
                seg = dst_st[sl] * R + typ_st[sl]
                plsc.addupdate_scatter(
                    hist,
                    [lax.shift_right_logical(seg, 7),
                     jnp.bitwise_and(seg, 127)],
                    ones16)
                return carry
            lax.fori_loop(0, PBLK // 16, suba, None)
        for p in range(5):
            for cc in range(8):
                rowi[p, pl.ds(cc * 16, 16)] = iota16 + (p * 128 + cc * 16)
        plsc.subcore_barrier()
        for p in range(5):
            pltpu.sync_copy(hist.at[pl.ds(p * 128, 128)],
                            deg_sh.at[rowi.at[p]], add=True)
        plsc.subcore_barrier()

        @pl.when(sid < 10)
        def _():
            r0 = sid * (DROWS // 10)
            pltpu.sync_copy(deg_sh.at[pl.ds(r0, DROWS // 10)],
                            dva.at[pl.ds(0, DROWS // 10)])
            for j in range(DROWS // 10):
                for cc in range(8):
                    sl = pl.ds(cc * 16, 16)
                    dva[j, sl] = 1.0 / jnp.maximum(dva[j, sl], 1.0)
            pltpu.sync_copy(dva.at[pl.ds(0, DROWS // 10)],
                            degh_h.at[pl.ds(r0, DROWS // 10)])
        plsc.subcore_barrier()

        NSB = PBLK // KB
        segs = (seg_a, seg_c)
        dvs = (dva, dvb)
        sds = (sd0, sd1)

        def bseg(i, b):
            e0 = i * KB
            for c in range(KB // 16):
                sl = pl.ds(e0 + c * 16, 16)
                seg = dst_st[sl] * R + typ_st[sl]
                segs[b][0, pl.ds(c * 16, 16)] = lax.shift_right_logical(
                    seg, 7)

        def start_dg(b):
            pltpu.make_async_copy(degh_h.at[segs[b].at[0]], dvs[b],
                                  sds[b]).start()

        def wait_dg(b):
            pltpu.make_async_copy(degh_h.at[segs[b].at[0]], dvs[b],
                                  sds[b]).wait()

        def wemit(i, b):
            e0 = i * KB
            for c in range(KB // 16):
                sl = pl.ds(e0 + c * 16, 16)
                seg = dst_st[sl] * R + typ_st[sl]
                w_st[sl] = plsc.load_gather(
                    dvs[b], [iota16 + c * 16, jnp.bitwise_and(seg, 127)])
                gi_st[sl] = typ_st[sl] * N + src_st[sl]

        for blk in range(EPT // PBLK):
            b0 = tbase + blk * PBLK
            pltpu.sync_copy(dst_h.at[pl.ds(b0, PBLK)], dst_st)
            pltpu.sync_copy(typ_h.at[pl.ds(b0, PBLK)], typ_st)
            pltpu.sync_copy(src_h.at[pl.ds(b0, PBLK)], src_st)

            bseg(0, 0)
            start_dg(0)

            def pairb(i2, carry):
                for b2 in (0, 1):
                    i = i2 * 2 + b2
                    nxt = i + 1

                    @pl.when(nxt <= NSB - 1)
                    def _():
                        bseg(nxt, 1 - b2)
                        start_dg(1 - b2)
                    wait_dg(b2)
                    wemit(i, b2)
                return carry
            lax.fori_loop(0, (NSB - 1) // 2, pairb, None)

            wait_dg(0)
            wemit(NSB - 1, 0)

            @pl.when(cid == 0)
            def _():
                pltpu.sync_copy(w_st, w_h.at[pl.ds(b0, PBLK)])
                pltpu.sync_copy(gi_st, gi_h.at[pl.ds(b0, PBLK)])

    return prep(src, dst, typ, z128)


def _sc_aggregate(gidx, dst, w, y, z128):
    mesh = plsc.VectorSubcoreMesh(core_axis_name="c", subcore_axis_name="s")
    slab = 624
    tail = N - NTILES * slab
    NSUB = PBLK // KB

    @functools.partial(
        pl.kernel,
        out_type=jax.ShapeDtypeStruct((2, N, HALF), jnp.float32),
        mesh=mesh,
        compiler_params=pltpu.CompilerParams(needs_layout_passes=False,
                                             use_tc_tiling_on_sc=True),
        scratch_types=[
            pltpu.VMEM((PBLK,), jnp.int32),
            pltpu.VMEM((PBLK,), jnp.int32),
            pltpu.VMEM((PBLK,), jnp.float32),
            pltpu.VMEM((2, KB), jnp.int32),
            pltpu.VMEM((2, KB), jnp.int32),
            pltpu.VMEM((2, KB, HALF), jnp.float32),
            pltpu.VMEM_SHARED((N, HALF), jnp.float32),
            pltpu.SemaphoreType.DMA,
            pltpu.SemaphoreType.DMA,
            pltpu.SemaphoreType.DMA,
            pltpu.SemaphoreType.DMA,
        ],
    )
    def agg(gi_h, dst_h, w_h, y_h, z128_h, out_h,
            gi_st, dst_st, w_st, gi_b, di_b, msg, acc_sh,
            sg0, sg1, ss0, ss1):
        cid = lax.axis_index("c")
        sid = lax.axis_index("s")
        tbase = sid * EPT
        sg = (sg0, sg1)
        ss = (ss0, ss1)

        r0 = sid * slab
        pltpu.sync_copy(z128_h.at[pl.ds(0, slab)], acc_sh.at[pl.ds(r0, slab)])

        @pl.when(sid == 0)
        def _():
            pltpu.sync_copy(z128_h.at[pl.ds(0, tail)],
                            acc_sh.at[pl.ds(NTILES * slab, tail)])
        plsc.subcore_barrier()

        coff = cid * (R * N)

        def build(i, b):
            e0 = i * KB
            for c in range(KB // 16):
                sl = pl.ds(e0 + c * 16, 16)
                sb = pl.ds(c * 16, 16)
                gi_b[b, sb] = gi_st[sl] + coff
                di_b[b, sb] = dst_st[sl]

        def start_gather(b):
            pltpu.make_async_copy(y_h.at[gi_b.at[b]], msg.at[b],
                                  sg[b]).start()

        def wait_gather(b):
            pltpu.make_async_copy(y_h.at[gi_b.at[b]], msg.at[b],
                                  sg[b]).wait()

        def start_scatter(b):
            pltpu.async_copy(msg.at[b], acc_sh.at[di_b.at[b]], ss[b],
                             add=True)

        def wait_scatter(b):
            pltpu.make_async_copy(msg.at[b], acc_sh.at[di_b.at[b]],
                                  ss[b]).wait()

        def scale(i, b):
            def srow(j, carry):
                idxv = lax.broadcast_in_dim(i * KB + j, (16,), ())
                wbj = plsc.load_gather(w_st, [idxv])
                for cc in range(HALF // 16):
                    s2 = pl.ds(cc * 16, 16)
                    msg[b, j, s2] = msg[b, j, s2] * wbj
                return carry
            lax.fori_loop(0, KB, srow, None)

        for mb in range(EPT // PBLK):
            b0 = tbase + mb * PBLK
            pltpu.sync_copy(gi_h.at[pl.ds(b0, PBLK)], gi_st)
            pltpu.sync_copy(dst_h.at[pl.ds(b0, PBLK)], dst_st)
            pltpu.sync_copy(w_h.at[pl.ds(b0, PBLK)], w_st)

            build(0, 0)
            start_gather(0)

            def pair(i2, carry):
                for b2 in (0, 1):
                    i = i2 * 2 + b2
                    nxt = i + 1

                    @pl.when(i >= 1)
                    def _():
                        wait_scatter(1 - b2)

                    @pl.when(nxt <= NSUB - 1)
                    def _():
                        build(nxt, 1 - b2)
                        start_gather(1 - b2)
                    wait_gather(b2)
                    scale(i, b2)
                    start_scatter(b2)
                return carry
            lax.fori_loop(0, (NSUB - 1) // 2, pair, None)

            wait_scatter(1)
            wait_gather(0)
            scale(NSUB - 1, 0)
            start_scatter(0)
            wait_scatter(0)

        plsc.subcore_barrier()
        pltpu.sync_copy(acc_sh.at[pl.ds(r0, slab)],
                        out_h.at[cid, pl.ds(r0, slab)])

        @pl.when(sid == 0)
        def _():
            pltpu.sync_copy(acc_sh.at[pl.ds(NTILES * slab, tail)],
                            out_h.at[cid, pl.ds(NTILES * slab, tail)])

    return agg(gidx, dst, w, y, z128)


def _tc_transform(h, wcomp, bases):
    def body(x_ref, wc_ref, b_ref, y_ref, wscr):
        i = pl.program_id(0)

        @pl.when(i == 0)
        def _():
            wc = wc_ref[...]
            ba = b_ref[...]
            for r in range(R):
                acc = wc[r, 0] * ba[0]
                for b in range(1, NB_BASES):
                    acc = acc + wc[r, b] * ba[b]
                wscr[r] = acc

        xb = x_ref[...]
        for r in range(R):
            y = jnp.dot(xb, wscr[r], preferred_element_type=jnp.float32)
            y_ref[0, r] = y[:, :HALF]
            y_ref[1, r] = y[:, HALF:]

    return pl.pallas_call(
        body,
        grid=(N // BN,),
        in_specs=[
            pl.BlockSpec((BN, D), lambda i: (i, 0)),
            pl.BlockSpec((R, NB_BASES), lambda i: (0, 0)),
            pl.BlockSpec((NB_BASES, D, H), lambda i: (0, 0, 0)),
        ],
        out_specs=pl.BlockSpec((2, R, BN, HALF), lambda i: (0, 0, i, 0)),
        out_shape=jax.ShapeDtypeStruct((2, R, N, HALF), jnp.float32),
        scratch_shapes=[pltpu.VMEM((R, D, H), jnp.float32)],
    )(h, wcomp, bases)


def _tc_finish(acc2, h, loop_w, bias2d):
    def body(a_ref, x_ref, l_ref, bias_ref, o_ref):
        a = jnp.concatenate([a_ref[0], a_ref[1]], axis=-1)
        out = a + jnp.dot(x_ref[...], l_ref[...],
                          preferred_element_type=jnp.float32) + bias_ref[...]
        o_ref[...] = jnp.maximum(out, 0.0)

    return pl.pallas_call(
        body,
        grid=(N // BN,),
        in_specs=[
            pl.BlockSpec((2, BN, HALF), lambda i: (0, i, 0)),
            pl.BlockSpec((BN, D), lambda i: (i, 0)),
            pl.BlockSpec((D, H), lambda i: (0, 0)),
            pl.BlockSpec((1, H), lambda i: (0, 0)),
        ],
        out_specs=pl.BlockSpec((BN, H), lambda i: (i, 0)),
        out_shape=jax.ShapeDtypeStruct((N, H), jnp.float32),
    )(acc2, h, loop_w, bias2d)


def kernel(x, edge_index, edge_type, bases1, wcomp1, loop1, bias1,
           bases2, wcomp2, loop2, bias2):
    src = edge_index[0]
    dst = edge_index[1]
    z128 = jnp.zeros((640, HALF), jnp.float32)
    w, gidx, _ = _sc_prep(src, dst, edge_type, z128)

    h = x
    for bases, wcomp, loop_w, bias in ((bases1, wcomp1, loop1, bias1),
                                       (bases2, wcomp2, loop2, bias2)):
        y = _tc_transform(h, wcomp, bases).reshape(2 * R * N, HALF)
        acc2 = _sc_aggregate(gidx, dst, w, y, z128)
        h = _tc_finish(acc2, h, loop_w, bias.reshape(1, H))
    return h


# speedup vs baseline: 5.6993x; 1.2245x over previous
import functools

import jax
import jax.numpy as jnp
from jax import lax
from jax.experimental import pallas as pl
from jax.experimental.pallas import tpu as pltpu
from jax.experimental.pallas import tpu_sc as plsc

N = 10000
E = 160000
D = 256
H = 256
R = 8
NB_BASES = 4
NR = N * R

HALF = 128
NTILES = 16
EPT = E // NTILES
PBLK = 2000
KB = 80
BN = 1000


def _sc_prep(src, dst, typ, z128):
    mesh = plsc.VectorSubcoreMesh(core_axis_name="c", subcore_axis_name="s")
    DROWS = 640

    @functools.partial(
        pl.kernel,
        out_type=(jax.ShapeDtypeStruct((E,), jnp.float32),
                  jax.ShapeDtypeStruct((E,), jnp.int32),
                  jax.ShapeDtypeStruct((DROWS, 128), jnp.float32)),
        mesh=mesh,
        compiler_params=pltpu.CompilerParams(needs_layout_passes=False,
                                             use_tc_tiling_on_sc=True),
        scratch_types=[
            pltpu.VMEM((PBLK,), jnp.int32),
            pltpu.VMEM((PBLK,), jnp.int32),
            pltpu.VMEM((PBLK,), jnp.int32),
            pltpu.VMEM((PBLK,), jnp.float32),
            pltpu.VMEM((PBLK,), jnp.int32),
            pltpu.VMEM((640, 128), jnp.float32),
            pltpu.VMEM((5, 128), jnp.int32),
            pltpu.VMEM_SHARED((640, 128), jnp.float32),
        ],
    )
    def prep(src_h, dst_h, typ_h, z128_h, w_h, gi_h, degh_h,
             dst_st, typ_st, src_st, w_st, gi_st, hist, rowi, deg_sh):
        cid = lax.axis_index("c")
        sid = lax.axis_index("s")
        tbase = sid * EPT
        iota16 = lax.iota(jnp.int32, 16)
        ones16 = jnp.ones((16,), jnp.float32)

        pltpu.sync_copy(z128_h, hist)

        @pl.when(sid < 10)
        def _():
            pltpu.sync_copy(z128_h.at[pl.ds(0, DROWS // 10)],
                            deg_sh.at[pl.ds(sid * (DROWS // 10),
                                            DROWS // 10)])

        for blk in range(EPT // PBLK):
            b0 = tbase + blk * PBLK
            pltpu.sync_copy(dst_h.at[pl.ds(b0, PBLK)], dst_st)
            pltpu.sync_copy(typ_h.at[pl.ds(b0, PBLK)], typ_st)

            def suba(k2, carry):
                sl = pl.ds(k2 * 16, 16)
                seg = dst_st[sl] * R + typ_st[sl]
                plsc.addupdate_scatter(
                    hist,
                    [lax.shift_right_logical(seg, 7),
                     jnp.bitwise_and(seg, 127)],
                    ones16)
                return carry
            lax.fori_loop(0, PBLK // 16, suba, None)
        for p in range(5):
            for cc in range(8):
                rowi[p, pl.ds(cc * 16, 16)] = iota16 + (p * 128 + cc * 16)
        plsc.subcore_barrier()
        for p in range(5):
            pltpu.sync_copy(hist.at[pl.ds(p * 128, 128)],
                            deg_sh.at[rowi.at[p]], add=True)
        plsc.subcore_barrier()

        pltpu.sync_copy(deg_sh, hist)

        def recip(j, carry):
            for cc in range(8):
                sl = pl.ds(cc * 16, 16)
                hist[j, sl] = 1.0 / jnp.maximum(hist[j, sl], 1.0)
            return carry
        lax.fori_loop(0, DROWS, recip, None)

        @pl.when(sid < 10)
        def _():
            r0 = sid * (DROWS // 10)
            pltpu.sync_copy(hist.at[pl.ds(r0, DROWS // 10)],
                            degh_h.at[pl.ds(r0, DROWS // 10)])

        for blk in range(EPT // PBLK):
            b0 = tbase + blk * PBLK
            pltpu.sync_copy(dst_h.at[pl.ds(b0, PBLK)], dst_st)
            pltpu.sync_copy(typ_h.at[pl.ds(b0, PBLK)], typ_st)
            pltpu.sync_copy(src_h.at[pl.ds(b0, PBLK)], src_st)

            def subb(k2, carry):
                sl = pl.ds(k2 * 16, 16)
                seg = dst_st[sl] * R + typ_st[sl]
                w_st[sl] = plsc.load_gather(
                    hist, [lax.shift_right_logical(seg, 7),
                           jnp.bitwise_and(seg, 127)])
                gi_st[sl] = typ_st[sl] * N + src_st[sl]
                return carry
            lax.fori_loop(0, PBLK // 16, subb, None)

            @pl.when(cid == 0)
            def _():
                pltpu.sync_copy(w_st, w_h.at[pl.ds(b0, PBLK)])
                pltpu.sync_copy(gi_st, gi_h.at[pl.ds(b0, PBLK)])

    return prep(src, dst, typ, z128)


def _sc_aggregate(gidx, dst, w, y, z128):
    mesh = plsc.VectorSubcoreMesh(core_axis_name="c", subcore_axis_name="s")
    slab = 624
    tail = N - NTILES * slab
    NSUB = PBLK // KB

    @functools.partial(
        pl.kernel,
        out_type=jax.ShapeDtypeStruct((2, N, HALF), jnp.float32),
        mesh=mesh,
        compiler_params=pltpu.CompilerParams(needs_layout_passes=False,
                                             use_tc_tiling_on_sc=True),
        scratch_types=[
            pltpu.VMEM((PBLK,), jnp.int32),
            pltpu.VMEM((PBLK,), jnp.int32),
            pltpu.VMEM((PBLK,), jnp.float32),
            pltpu.VMEM((2, KB), jnp.int32),
            pltpu.VMEM((2, KB), jnp.int32),
            pltpu.VMEM((2, KB, HALF), jnp.float32),
            pltpu.VMEM_SHARED((N, HALF), jnp.float32),
            pltpu.SemaphoreType.DMA,
            pltpu.SemaphoreType.DMA,
            pltpu.SemaphoreType.DMA,
            pltpu.SemaphoreType.DMA,
        ],
    )
    def agg(gi_h, dst_h, w_h, y_h, z128_h, out_h,
            gi_st, dst_st, w_st, gi_b, di_b, msg, acc_sh,
            sg0, sg1, ss0, ss1):
        cid = lax.axis_index("c")
        sid = lax.axis_index("s")
        tbase = sid * EPT
        sg = (sg0, sg1)
        ss = (ss0, ss1)

        r0 = sid * slab
        pltpu.sync_copy(z128_h.at[pl.ds(0, slab)], acc_sh.at[pl.ds(r0, slab)])

        @pl.when(sid == 0)
        def _():
            pltpu.sync_copy(z128_h.at[pl.ds(0, tail)],
                            acc_sh.at[pl.ds(NTILES * slab, tail)])
        plsc.subcore_barrier()

        coff = cid * (R * N)

        def build(i, b):
            e0 = i * KB
            for c in range(KB // 16):
                sl = pl.ds(e0 + c * 16, 16)
                sb = pl.ds(c * 16, 16)
                gi_b[b, sb] = gi_st[sl] + coff
                di_b[b, sb] = dst_st[sl]

        def start_gather(b):
            pltpu.make_async_copy(y_h.at[gi_b.at[b]], msg.at[b],
                                  sg[b]).start()

        def wait_gather(b):
            pltpu.make_async_copy(y_h.at[gi_b.at[b]], msg.at[b],
                                  sg[b]).wait()

        def start_scatter(b):
            pltpu.async_copy(msg.at[b], acc_sh.at[di_b.at[b]], ss[b],
                             add=True)

        def wait_scatter(b):
            pltpu.make_async_copy(msg.at[b], acc_sh.at[di_b.at[b]],
                                  ss[b]).wait()

        def scale(i, b):
            def srow(j, carry):
                idxv = lax.broadcast_in_dim(i * KB + j, (16,), ())
                wbj = plsc.load_gather(w_st, [idxv])
                for cc in range(HALF // 16):
                    s2 = pl.ds(cc * 16, 16)
                    msg[b, j, s2] = msg[b, j, s2] * wbj
                return carry
            lax.fori_loop(0, KB, srow, None)

        for mb in range(EPT // PBLK):
            b0 = tbase + mb * PBLK
            pltpu.sync_copy(gi_h.at[pl.ds(b0, PBLK)], gi_st)
            pltpu.sync_copy(dst_h.at[pl.ds(b0, PBLK)], dst_st)
            pltpu.sync_copy(w_h.at[pl.ds(b0, PBLK)], w_st)

            build(0, 0)
            start_gather(0)

            def pair(i2, carry):
                for b2 in (0, 1):
                    i = i2 * 2 + b2
                    nxt = i + 1

                    @pl.when(i >= 1)
                    def _():
                        wait_scatter(1 - b2)

                    @pl.when(nxt <= NSUB - 1)
                    def _():
                        build(nxt, 1 - b2)
                        start_gather(1 - b2)
                    wait_gather(b2)
                    scale(i, b2)
                    start_scatter(b2)
                return carry
            lax.fori_loop(0, (NSUB - 1) // 2, pair, None)

            wait_scatter(1)
            wait_gather(0)
            scale(NSUB - 1, 0)
            start_scatter(0)
            wait_scatter(0)

        plsc.subcore_barrier()
        pltpu.sync_copy(acc_sh.at[pl.ds(r0, slab)],
                        out_h.at[cid, pl.ds(r0, slab)])

        @pl.when(sid == 0)
        def _():
            pltpu.sync_copy(acc_sh.at[pl.ds(NTILES * slab, tail)],
                            out_h.at[cid, pl.ds(NTILES * slab, tail)])

    return agg(gidx, dst, w, y, z128)


def _tc_transform(h, wcomp, bases):
    def body(x_ref, wc_ref, b_ref, y_ref, wscr):
        i = pl.program_id(0)

        @pl.when(i == 0)
        def _():
            wc = wc_ref[...]
            ba = b_ref[...]
            for r in range(R):
                acc = wc[r, 0] * ba[0]
                for b in range(1, NB_BASES):
                    acc = acc + wc[r, b] * ba[b]
                wscr[r] = acc

        xb = x_ref[...]
        for r in range(R):
            y = jnp.dot(xb, wscr[r], preferred_element_type=jnp.float32)
            y_ref[0, r] = y[:, :HALF]
            y_ref[1, r] = y[:, HALF:]

    return pl.pallas_call(
        body,
        grid=(N // BN,),
        in_specs=[
            pl.BlockSpec((BN, D), lambda i: (i, 0)),
            pl.BlockSpec((R, NB_BASES), lambda i: (0, 0)),
            pl.BlockSpec((NB_BASES, D, H), lambda i: (0, 0, 0)),
        ],
        out_specs=pl.BlockSpec((2, R, BN, HALF), lambda i: (0, 0, i, 0)),
        out_shape=jax.ShapeDtypeStruct((2, R, N, HALF), jnp.float32),
        scratch_shapes=[pltpu.VMEM((R, D, H), jnp.float32)],
    )(h, wcomp, bases)


def _tc_finish(acc2, h, loop_w, bias2d):
    def body(a_ref, x_ref, l_ref, bias_ref, o_ref):
        a = jnp.concatenate([a_ref[0], a_ref[1]], axis=-1)
        out = a + jnp.dot(x_ref[...], l_ref[...],
                          preferred_element_type=jnp.float32) + bias_ref[...]
        o_ref[...] = jnp.maximum(out, 0.0)

    return pl.pallas_call(
        body,
        grid=(N // BN,),
        in_specs=[
            pl.BlockSpec((2, BN, HALF), lambda i: (0, i, 0)),
            pl.BlockSpec((BN, D), lambda i: (i, 0)),
            pl.BlockSpec((D, H), lambda i: (0, 0)),
            pl.BlockSpec((1, H), lambda i: (0, 0)),
        ],
        out_specs=pl.BlockSpec((BN, H), lambda i: (i, 0)),
        out_shape=jax.ShapeDtypeStruct((N, H), jnp.float32),
    )(acc2, h, loop_w, bias2d)


def kernel(x, edge_index, edge_type, bases1, wcomp1, loop1, bias1,
           bases2, wcomp2, loop2, bias2):
    src = edge_index[0]
    dst = edge_index[1]
    z128 = jnp.zeros((640, HALF), jnp.float32)
    w, gidx, _ = _sc_prep(src, dst, edge_type, z128)

    h = x
    for bases, wcomp, loop_w, bias in ((bases1, wcomp1, loop1, bias1),
                                       (bases2, wcomp2, loop2, bias2)):
        y = _tc_transform(h, wcomp, bases).reshape(2 * R * N, HALF)
        acc2 = _sc_aggregate(gidx, dst, w, y, z128)
        h = _tc_finish(acc2, h, loop_w, bias.reshape(1, H))
    return h
